# baseline (speedup 1.0000x reference)
import functools

import jax
import jax.numpy as jnp
from jax import lax
from jax.experimental import pallas as pl
from jax.experimental.pallas import tpu as pltpu
from jax.experimental.pallas import tpu_sc as plsc

_L = 16
_NW = 32
_B = 16384
_BPW = _B // _NW
_LEN = 100000
_D = 64
_NB = 4


def _body(inp_hbm, gt_hbm, xs_hbm, outx_hbm, outg_hbm,
          xs_v, idx0, idx1, idx2, idx3, rows_a, rows_b, sem_g, sem_w):
    wid = lax.axis_index("s") * 2 + lax.axis_index("c")
    base = wid * _BPW
    idxs = (idx0, idx1, idx2, idx3)

    pltpu.sync_copy(xs_hbm.at[pl.ds(base, _BPW)], xs_v)
    for i in range(_BPW // _L):
        v = xs_v[pl.ds(i * _L, _L)]
        ii = (v * jnp.float32(_LEN)).astype(jnp.int32)
        ii = jnp.minimum(jnp.maximum(ii, jnp.int32(0)), jnp.int32(_LEN - 1))
        for b in range(_NB):
            idxs[b][pl.ds(i * _L, _L)] = ii + jnp.int32(b * _LEN)

    steps = [(tab, out, b)
             for tab, out in ((inp_hbm, outx_hbm), (gt_hbm, outg_hbm))
             for b in range(_NB)]
    bufs = (rows_a, rows_b)

    tab0, _, b0 = steps[0]
    pltpu.async_copy(tab0.at[idxs[b0]], bufs[0], sem_g).wait()
    for k, (tab, out, b) in enumerate(steps):
        cur = bufs[k % 2]
        if k + 1 < len(steps):
            ntab, _, nb = steps[k + 1]
            gather = pltpu.async_copy(ntab.at[idxs[nb]], bufs[(k + 1) % 2],
                                      sem_g)
        write = pltpu.async_copy(cur, out.at[pl.ds(b * _B + base, _BPW)],
                                 sem_w)
        if k + 1 < len(steps):
            gather.wait()
        write.wait()


@jax.jit
def kernel(inp, gt, xs):
    inp_flat = inp.reshape(_NB * _LEN, _D)
    gt_flat = gt.reshape(_NB * _LEN, _D)
    mesh = plsc.VectorSubcoreMesh(core_axis_name="c", subcore_axis_name="s")
    out_type = (jax.ShapeDtypeStruct((_NB * _B, _D), jnp.float32),
                jax.ShapeDtypeStruct((_NB * _B, _D), jnp.float32))
    run = pl.kernel(
        _body,
        out_type=out_type,
        mesh=mesh,
        scratch_types=[
            pltpu.VMEM((_BPW,), jnp.float32),
            pltpu.VMEM((_BPW,), jnp.int32),
            pltpu.VMEM((_BPW,), jnp.int32),
            pltpu.VMEM((_BPW,), jnp.int32),
            pltpu.VMEM((_BPW,), jnp.int32),
            pltpu.VMEM((_BPW, _D), jnp.float32),
            pltpu.VMEM((_BPW, _D), jnp.float32),
            pltpu.SemaphoreType.DMA,
            pltpu.SemaphoreType.DMA,
        ],
        compiler_params=pltpu.CompilerParams(use_tc_tiling_on_sc=False),
    )
    return run(inp_flat, gt_flat, xs)
